# baseline (speedup 1.0000x reference)
import functools
import math

import jax
import jax.numpy as jnp
from jax import lax
from jax.experimental import pallas as pl
from jax.experimental.pallas import tpu as pltpu
from jax.experimental.pallas import tpu_sc as plsc

NC = 2
NS = 16
EB = 48


def _ru(x, m):
    return (x + m - 1) // m * m


def _gelu(x):
    return 0.5 * x * (1.0 + lax.erf(x * (1.0 / math.sqrt(2.0))))


def _qkv_body(h_ref, w1_ref, b1_ref, w2_ref, b2_ref, wq_ref, wk_ref, wv_ref,
              q_ref, k_ref, v_ref):
    h = h_ref[...]
    m = _gelu(jnp.dot(h, w1_ref[...], preferred_element_type=jnp.float32)
              + b1_ref[...])
    m = _gelu(jnp.dot(m, w2_ref[...], preferred_element_type=jnp.float32)
              + b2_ref[...])
    q_ref[...] = jnp.dot(h, wq_ref[...], preferred_element_type=jnp.float32)
    k_ref[...] = jnp.dot(h, wk_ref[...], preferred_element_type=jnp.float32)
    v_ref[...] = jnp.dot(m, wv_ref[...], preferred_element_type=jnp.float32)


def _qkv(hn, msg_W1, msg_b1, msg_W2, msg_b2, Wq, Wk, Wv, blk):
    tt, dh = hn.shape
    grid = (tt // blk,)
    full = lambda s: pl.BlockSpec(s, lambda i: (0, 0))
    rows = pl.BlockSpec((blk, dh), lambda i: (i, 0))
    return pl.pallas_call(
        _qkv_body,
        grid=grid,
        in_specs=[rows, full((dh, dh)), full((1, dh)), full((dh, dh)),
                  full((1, dh)), full((dh, dh)), full((dh, dh)),
                  full((dh, dh))],
        out_specs=[rows, rows, rows],
        out_shape=[jax.ShapeDtypeStruct((tt, dh), jnp.float32)] * 3,
    )(hn, msg_W1, msg_b1.reshape(1, dh), msg_W2, msg_b2.reshape(1, dh),
      Wq, Wk, Wv)


def _sc_edge(qtab, kvtab, meta, wdat, ha, hd, n_heads, d_k, blocks_per_w,
             n_passes):
    dh = qtab.shape[1]
    rp = ha // NS
    rpd = hd // NS
    inv_sqrt = 1.0 / math.sqrt(d_k)
    mesh = plsc.VectorSubcoreMesh(core_axis_name="c", subcore_axis_name="s",
                                  num_cores=NC, num_subcores=NS)

    @functools.partial(
        pl.kernel,
        out_type=[jax.ShapeDtypeStruct((NC, n_passes, ha, dh), jnp.float32),
                  jax.ShapeDtypeStruct((NC, n_passes, hd, dh), jnp.float32)],
        mesh=mesh,
        compiler_params=pltpu.CompilerParams(needs_layout_passes=False),
        scratch_types=[
            pltpu.VMEM((2 + 2 * n_passes, EB), jnp.int32),
            pltpu.VMEM((2 + 2 * n_passes, EB), jnp.int32),
            pltpu.VMEM((1 + n_passes, EB), jnp.float32),
            pltpu.VMEM((1 + n_passes, EB), jnp.float32),
            pltpu.VMEM((EB, dh), jnp.float32),
            pltpu.VMEM((EB, dh), jnp.float32),
            pltpu.VMEM((EB, 2 * dh), jnp.float32),
            pltpu.VMEM((EB, 2 * dh), jnp.float32),
            pltpu.VMEM((EB, dh), jnp.float32),
            pltpu.VMEM_SHARED((ha, dh), jnp.float32),
            pltpu.VMEM_SHARED((hd, dh), jnp.float32),
            pltpu.SemaphoreType.DMA,
            pltpu.SemaphoreType.DMA,
            pltpu.SemaphoreType.DMA,
            pltpu.SemaphoreType.DMA,
        ],
    )
    def body(qtab_h, kvtab_h, meta_h, wdat_h, znum_h, num_out, den_out,
             meta_vA, meta_vB, wdat_vA, wdat_vB, qbA, qbB, kvbA, kvbB, pb,
             num_acc, den_acc, semA0, semA1, semB0, semB1):
        c = lax.axis_index("c")
        s = lax.axis_index("s")
        pltpu.sync_copy(znum_h.at[pl.ds(0, EB)], pb)
        lane = lax.iota(jnp.int32, 16)

        bufs = ((meta_vA, wdat_vA, qbA, kvbA, semA0, semA1),
                (meta_vB, wdat_vB, qbB, kvbB, semB0, semB1))

        def issue(blk, bi):
            meta_v, wdat_v, qb, kvb, sem0, sem1 = bufs[bi]
            pltpu.sync_copy(meta_h.at[c, blk], meta_v)
            pltpu.sync_copy(wdat_h.at[c, blk], wdat_v)
            cp0 = pltpu.async_copy(qtab_h.at[meta_v.at[0]], qb, sem0)
            cp1 = pltpu.async_copy(kvtab_h.at[meta_v.at[1]], kvb, sem1)
            return cp0, cp1

        def wait(bi):
            meta_v, wdat_v, qb, kvb, sem0, sem1 = bufs[bi]
            pltpu.make_async_copy(qtab_h.at[meta_v.at[0]], qb, sem0).wait()
            pltpu.make_async_copy(kvtab_h.at[meta_v.at[1]], kvb, sem1).wait()

        def process(half, bi):
            meta_v, wdat_v, qb, kvb, sem0, sem1 = bufs[bi]

            def group_body(g, _):
                erow = lane + g * 16
                wvec = plsc.load_gather(wdat_v,
                                        [jnp.zeros((16,), jnp.int32), erow])
                cb = plsc.load_gather(wdat_v,
                                      [jnp.full((16,), 1 + half, jnp.int32),
                                       erow]).astype(jnp.int32)
                wclip = jnp.maximum(wvec, 1e-10)
                ps = []
                for h in range(n_heads):
                    def dot_f(f, acc):
                        fv = jnp.full((16,), h * d_k, jnp.int32) + f
                        qv = plsc.load_gather(qb, [erow, fv])
                        kv = plsc.load_gather(kvb, [erow, fv])
                        return acc + qv * kv
                    acc = lax.fori_loop(0, d_k, dot_f,
                                        jnp.zeros((16,), jnp.float32))
                    p = wclip * jnp.exp(acc * inv_sqrt)
                    ps.append(p)
                    plsc.store_scatter(pb, [erow, cb + h], p)
                for h in range(n_heads):
                    def wv_f(f, _):
                        fv = jnp.full((16,), h * d_k, jnp.int32) + f
                        vv = plsc.load_gather(kvb, [erow, fv + dh])
                        plsc.store_scatter(qb, [erow, fv], vv * ps[h])
                        return 0
                    lax.fori_loop(0, d_k, wv_f, 0)
                return 0

            lax.fori_loop(0, EB // 16, group_body, 0)
            pltpu.sync_copy(qb, num_acc.at[meta_v.at[2 + half]], add=True)
            pltpu.sync_copy(pb, den_acc.at[meta_v.at[2 + n_passes + half]],
                            add=True)

            def zero_body(g, _):
                erow = lane + g * 16
                cb = plsc.load_gather(wdat_v,
                                      [jnp.full((16,), 1 + half, jnp.int32),
                                       erow]).astype(jnp.int32)
                zz = jnp.zeros((16,), jnp.float32)
                for h in range(n_heads):
                    plsc.store_scatter(pb, [erow, cb + h], zz)
                return 0

            lax.fori_loop(0, EB // 16, zero_body, 0)

        assert blocks_per_w % 2 == 1
        npairs = (blocks_per_w - 1) // 2
        for half in range(n_passes):
            pltpu.sync_copy(znum_h.at[pl.ds(s * rp, rp)],
                            num_acc.at[pl.ds(s * rp, rp)])
            pltpu.sync_copy(znum_h.at[pl.ds(s * rpd, rpd)],
                            den_acc.at[pl.ds(s * rpd, rpd)])
            plsc.subcore_barrier()
            base = s * blocks_per_w

            issue(base, 0)

            def pair_body(i, _):
                b = 2 * i
                wait(0)
                issue(base + b + 1, 1)
                process(half, 0)
                wait(1)
                issue(base + b + 2, 0)
                process(half, 1)
                return 0

            lax.fori_loop(0, npairs, pair_body, 0)
            wait(0)
            process(half, 0)
            plsc.subcore_barrier()
            pltpu.sync_copy(num_acc.at[pl.ds(s * rp, rp)],
                            num_out.at[c, half, pl.ds(s * rp, rp)])
            pltpu.sync_copy(den_acc.at[pl.ds(s * rpd, rpd)],
                            den_out.at[c, half, pl.ds(s * rpd, rpd)])

    znum = jnp.zeros((ha, dh), jnp.float32)
    return body(qtab, kvtab, meta, wdat, znum)


def _upd_body(h_ref, num_ref, den_ref, w1a_ref, w1b_ref, b1_ref, w2_ref,
              b2_ref, out_ref, *, n_heads, d_k):
    den16 = den_ref[...]
    heads = lax.broadcasted_iota(jnp.int32, (16, n_heads * d_k), 0)
    cols = lax.broadcasted_iota(jnp.int32, (16, n_heads * d_k), 1) // d_k
    expand = (heads == cols).astype(jnp.float32)
    den_exp = jnp.dot(den16, expand, preferred_element_type=jnp.float32)
    msg = num_ref[...] / (den_exp + 1e-10)
    h = h_ref[...]
    h1 = _gelu(jnp.dot(h, w1a_ref[...], preferred_element_type=jnp.float32)
               + jnp.dot(msg, w1b_ref[...], preferred_element_type=jnp.float32)
               + b1_ref[...])
    out_ref[...] = _gelu(jnp.dot(h1, w2_ref[...],
                                 preferred_element_type=jnp.float32)
                         + b2_ref[...])


def _update(hn, num, den, upd_W1, upd_b1, upd_W2, upd_b2, n_heads, d_k, blk):
    tt, dh = hn.shape
    grid = (tt // blk,)
    full = lambda s: pl.BlockSpec(s, lambda i: (0, 0))
    rows = pl.BlockSpec((blk, dh), lambda i: (i, 0))
    rows16 = pl.BlockSpec((blk, 16), lambda i: (i, 0))
    w1a = upd_W1[:dh]
    w1b = upd_W1[dh:]
    return pl.pallas_call(
        functools.partial(_upd_body, n_heads=n_heads, d_k=d_k),
        grid=grid,
        in_specs=[rows, rows, rows16, full((dh, dh)), full((dh, dh)),
                  full((1, dh)), full((dh, dh)), full((1, dh))],
        out_specs=rows,
        out_shape=jax.ShapeDtypeStruct((tt, dh), jnp.float32),
    )(hn, num, den, w1a, w1b, upd_b1.reshape(1, dh), upd_W2,
      upd_b2.reshape(1, dh))


def kernel(inv_h, asset_h, edge_index, inv_norm_w, asset_norm_w,
           num_investors, num_assets,
           msg_W1, msg_b1, msg_W2, msg_b2, Wq, Wk, Wv,
           upd_W1, upd_b1, upd_W2, upd_b2):
    n_i, dh = inv_h.shape
    n_a = asset_h.shape[0]
    n_e = edge_index.shape[1]
    hd = Wq.shape[1]
    d_k = 32
    n_heads = hd // d_k

    roff = _ru(max(n_i, n_a) + 1, NS * 8)
    tt = _ru(2 * roff, 1024)
    hn = jnp.zeros((tt, dh), jnp.float32)
    hn = hn.at[:n_i].set(inv_h).at[roff:roff + n_a].set(asset_h)

    pe = _ru(n_e, NS * EB)
    pad = pe - n_e
    tgt = edge_index[0].astype(jnp.int32)
    src = edge_index[1].astype(jnp.int32)
    tgt_p = jnp.concatenate([tgt, jnp.full((pad,), n_i, jnp.int32)])
    src_p = jnp.concatenate([src, jnp.full((pad,), n_a, jnp.int32)])
    srco_p = src_p + roff
    wi_p = jnp.concatenate([inv_norm_w, jnp.zeros((pad,), jnp.float32)])
    wa_p = jnp.concatenate([asset_norm_w, jnp.zeros((pad,), jnp.float32)])

    NP = 4
    hs = roff // NP
    ha = _ru(hs + 1, NS * 8)
    hd = _ru(hs // 8 + 1, NS * 8)
    out0 = jnp.stack([tgt_p, src_p])
    ohs, drows, dcbs = [], [], []
    for k in range(NP):
        inr = (out0 >= k * hs) & (out0 < (k + 1) * hs)
        tk = out0 - k * hs
        ohs.append(jnp.where(inr, tk, hs).reshape(NC, -1, 1, EB))
        drows.append(jnp.where(inr, tk // 8, hd - 1).reshape(NC, -1, 1, EB))
        dcbs.append(jnp.where(inr, (tk % 8) * 16, 0)
                    .astype(jnp.float32).reshape(NC, -1, 1, EB))

    nblk = pe // EB
    qidx = jnp.stack([tgt_p, srco_p]).reshape(NC, nblk, 1, EB)
    kvidx = jnp.stack([srco_p, tgt_p]).reshape(NC, nblk, 1, EB)
    meta = jnp.concatenate([qidx, kvidx] + ohs + drows, axis=2)
    wdat = jnp.concatenate(
        [jnp.stack([wi_p, wa_p]).reshape(NC, nblk, 1, EB)] + dcbs, axis=2)

    q, k, v = _qkv(hn, msg_W1, msg_b1, msg_W2, msg_b2, Wq, Wk, Wv, blk=1024)
    kv = jnp.concatenate([k, v], axis=1)

    num, den = _sc_edge(q, kv, meta, wdat, ha, hd, n_heads, d_k,
                        blocks_per_w=nblk // NS, n_passes=NP)

    numf = jnp.zeros((tt, dh), jnp.float32)
    denf = jnp.zeros((tt, 16), jnp.float32)
    for c in range(NC):
        base = c * roff
        for k in range(NP):
            den16 = den[c, k, :hs // 8].reshape(hs, 16)
            numf = numf.at[base + k * hs:base + (k + 1) * hs].set(
                num[c, k, :hs])
            denf = denf.at[base + k * hs:base + (k + 1) * hs].set(den16)

    out = _update(hn, numf, denf, upd_W1, upd_b1, upd_W2, upd_b2,
                  n_heads, d_k, blk=1024)
    return out[:n_i], out[roff:roff + n_a]


# speedup vs baseline: 1.9235x; 1.9235x over previous
import functools
import math

import jax
import jax.numpy as jnp
from jax import lax
from jax.experimental import pallas as pl
from jax.experimental.pallas import tpu as pltpu
from jax.experimental.pallas import tpu_sc as plsc

NC = 2
NS = 16
EB = 64


def _ru(x, m):
    return (x + m - 1) // m * m


def _gelu(x):
    return 0.5 * x * (1.0 + lax.erf(x * (1.0 / math.sqrt(2.0))))


def _qkv_body(h_ref, w1_ref, b1_ref, w2_ref, b2_ref, wq_ref, wk_ref, wv_ref,
              q_ref, k_ref, v_ref):
    h = h_ref[...]
    m = _gelu(jnp.dot(h, w1_ref[...], preferred_element_type=jnp.float32)
              + b1_ref[...])
    m = _gelu(jnp.dot(m, w2_ref[...], preferred_element_type=jnp.float32)
              + b2_ref[...])
    q_ref[...] = jnp.dot(h, wq_ref[...], preferred_element_type=jnp.float32)
    k_ref[...] = jnp.dot(h, wk_ref[...], preferred_element_type=jnp.float32)
    v_ref[...] = jnp.dot(m, wv_ref[...], preferred_element_type=jnp.float32)


def _qkv(hn, msg_W1, msg_b1, msg_W2, msg_b2, Wq, Wk, Wv, blk):
    tt, dh = hn.shape
    grid = (tt // blk,)
    full = lambda s: pl.BlockSpec(s, lambda i: (0, 0))
    rows = pl.BlockSpec((blk, dh), lambda i: (i, 0))
    return pl.pallas_call(
        _qkv_body,
        grid=grid,
        in_specs=[rows, full((dh, dh)), full((1, dh)), full((dh, dh)),
                  full((1, dh)), full((dh, dh)), full((dh, dh)),
                  full((dh, dh))],
        out_specs=[rows, rows, rows],
        out_shape=[jax.ShapeDtypeStruct((tt, dh), jnp.float32)] * 3,
    )(hn, msg_W1, msg_b1.reshape(1, dh), msg_W2, msg_b2.reshape(1, dh),
      Wq, Wk, Wv)


def _sc_edge(qtab, kvtab, meta, wdat, ha, hd, n_heads, d_k, blocks_per_w,
             n_passes):
    dh = qtab.shape[1]
    rp = ha // NS
    rpd = hd // NS
    inv_sqrt = 1.0 / math.sqrt(d_k)
    mesh = plsc.VectorSubcoreMesh(core_axis_name="c", subcore_axis_name="s",
                                  num_cores=NC, num_subcores=NS)

    @functools.partial(
        pl.kernel,
        out_type=[jax.ShapeDtypeStruct((NC, n_passes, ha, dh), jnp.float32),
                  jax.ShapeDtypeStruct((NC, n_passes, hd, dh), jnp.float32)],
        mesh=mesh,
        compiler_params=pltpu.CompilerParams(needs_layout_passes=False),
        scratch_types=[
            pltpu.VMEM((2 + 2 * n_passes, EB), jnp.int32),
            pltpu.VMEM((1 + n_passes, EB), jnp.float32),
            pltpu.VMEM((EB, dh), jnp.float32),
            pltpu.VMEM((EB, 2 * dh), jnp.float32),
            pltpu.VMEM((EB, dh), jnp.float32),
            pltpu.VMEM_SHARED((ha, dh), jnp.float32),
            pltpu.VMEM_SHARED((hd, dh), jnp.float32),
            pltpu.SemaphoreType.DMA,
            pltpu.SemaphoreType.DMA,
            pltpu.SemaphoreType.DMA,
            pltpu.SemaphoreType.DMA,
            pltpu.SemaphoreType.DMA,
            pltpu.SemaphoreType.DMA,
        ],
    )
    def body(qtab_h, kvtab_h, meta_h, wdat_h, znum_h, num_out, den_out,
             meta_v, wdat_v, qb, kvb, pb, num_acc, den_acc, sem0, sem1,
             semm0, semm1, sems0, sems1):
        c = lax.axis_index("c")
        s = lax.axis_index("s")
        pltpu.sync_copy(znum_h.at[pl.ds(0, EB)], pb)
        lane = lax.iota(jnp.int32, 16)

        for half in range(n_passes):
            pltpu.sync_copy(znum_h.at[pl.ds(s * rp, rp)],
                            num_acc.at[pl.ds(s * rp, rp)])
            pltpu.sync_copy(znum_h.at[pl.ds(s * rpd, rpd)],
                            den_acc.at[pl.ds(s * rpd, rpd)])
            plsc.subcore_barrier()

            def block_body(b, _):
                blk = s * blocks_per_w + b
                cpm = pltpu.async_copy(meta_h.at[c, blk], meta_v, semm0)
                cpw = pltpu.async_copy(wdat_h.at[c, blk], wdat_v, semm1)
                cpm.wait()
                cpw.wait()
                cp0 = pltpu.async_copy(qtab_h.at[meta_v.at[0]], qb, sem0)
                cp1 = pltpu.async_copy(kvtab_h.at[meta_v.at[1]], kvb, sem1)
                cp0.wait()
                cp1.wait()

                def group_body(g, _):
                    erow = lane + g * 16
                    wvec = plsc.load_gather(wdat_v,
                                            [jnp.zeros((16,), jnp.int32),
                                             erow])
                    cb = plsc.load_gather(wdat_v,
                                          [jnp.full((16,), 1 + half,
                                                    jnp.int32), erow]
                                          ).astype(jnp.int32)
                    wclip = jnp.maximum(wvec, 1e-10)
                    ps = []
                    for h in range(n_heads):
                        def dot_f(f, acc):
                            fv = jnp.full((16,), h * d_k, jnp.int32) + f
                            qv = plsc.load_gather(qb, [erow, fv])
                            kv = plsc.load_gather(kvb, [erow, fv])
                            return acc + qv * kv
                        acc = lax.fori_loop(0, d_k, dot_f,
                                            jnp.zeros((16,), jnp.float32))
                        p = wclip * jnp.exp(acc * inv_sqrt)
                        ps.append(p)
                        plsc.store_scatter(pb, [erow, cb + h], p)
                    for h in range(n_heads):
                        def wv_f(f, _):
                            fv = jnp.full((16,), h * d_k, jnp.int32) + f
                            vv = plsc.load_gather(kvb, [erow, fv + dh])
                            plsc.store_scatter(qb, [erow, fv], vv * ps[h])
                            return 0
                        lax.fori_loop(0, d_k, wv_f, 0)
                    return 0

                lax.fori_loop(0, EB // 16, group_body, 0)
                cpn = pltpu.async_copy(qb, num_acc.at[meta_v.at[2 + half]],
                                       sems0, add=True)
                cpd = pltpu.async_copy(
                    pb, den_acc.at[meta_v.at[2 + n_passes + half]], sems1,
                    add=True)
                cpn.wait()
                cpd.wait()

                def zero_body(g, _):
                    erow = lane + g * 16
                    cb = plsc.load_gather(wdat_v,
                                          [jnp.full((16,), 1 + half,
                                                    jnp.int32), erow]
                                          ).astype(jnp.int32)
                    zz = jnp.zeros((16,), jnp.float32)
                    for h in range(n_heads):
                        plsc.store_scatter(pb, [erow, cb + h], zz)
                    return 0

                lax.fori_loop(0, EB // 16, zero_body, 0)
                return 0

            lax.fori_loop(0, blocks_per_w, block_body, 0)
            plsc.subcore_barrier()
            pltpu.sync_copy(num_acc.at[pl.ds(s * rp, rp)],
                            num_out.at[c, half, pl.ds(s * rp, rp)])
            pltpu.sync_copy(den_acc.at[pl.ds(s * rpd, rpd)],
                            den_out.at[c, half, pl.ds(s * rpd, rpd)])

    znum = jnp.zeros((ha, dh), jnp.float32)
    return body(qtab, kvtab, meta, wdat, znum)


def _upd_body(h_ref, num_ref, den_ref, w1a_ref, w1b_ref, b1_ref, w2_ref,
              b2_ref, out_ref, *, n_heads, d_k):
    den16 = den_ref[...]
    heads = lax.broadcasted_iota(jnp.int32, (16, n_heads * d_k), 0)
    cols = lax.broadcasted_iota(jnp.int32, (16, n_heads * d_k), 1) // d_k
    expand = (heads == cols).astype(jnp.float32)
    den_exp = jnp.dot(den16, expand, preferred_element_type=jnp.float32)
    msg = num_ref[...] / (den_exp + 1e-10)
    h = h_ref[...]
    h1 = _gelu(jnp.dot(h, w1a_ref[...], preferred_element_type=jnp.float32)
               + jnp.dot(msg, w1b_ref[...], preferred_element_type=jnp.float32)
               + b1_ref[...])
    out_ref[...] = _gelu(jnp.dot(h1, w2_ref[...],
                                 preferred_element_type=jnp.float32)
                         + b2_ref[...])


def _update(hn, num, den, upd_W1, upd_b1, upd_W2, upd_b2, n_heads, d_k, blk):
    tt, dh = hn.shape
    grid = (tt // blk,)
    full = lambda s: pl.BlockSpec(s, lambda i: (0, 0))
    rows = pl.BlockSpec((blk, dh), lambda i: (i, 0))
    rows16 = pl.BlockSpec((blk, 16), lambda i: (i, 0))
    w1a = upd_W1[:dh]
    w1b = upd_W1[dh:]
    return pl.pallas_call(
        functools.partial(_upd_body, n_heads=n_heads, d_k=d_k),
        grid=grid,
        in_specs=[rows, rows, rows16, full((dh, dh)), full((dh, dh)),
                  full((1, dh)), full((dh, dh)), full((1, dh))],
        out_specs=rows,
        out_shape=jax.ShapeDtypeStruct((tt, dh), jnp.float32),
    )(hn, num, den, w1a, w1b, upd_b1.reshape(1, dh), upd_W2,
      upd_b2.reshape(1, dh))


def kernel(inv_h, asset_h, edge_index, inv_norm_w, asset_norm_w,
           num_investors, num_assets,
           msg_W1, msg_b1, msg_W2, msg_b2, Wq, Wk, Wv,
           upd_W1, upd_b1, upd_W2, upd_b2):
    n_i, dh = inv_h.shape
    n_a = asset_h.shape[0]
    n_e = edge_index.shape[1]
    hd = Wq.shape[1]
    d_k = 32
    n_heads = hd // d_k

    roff = _ru(max(n_i, n_a) + 1, NS * 8)
    tt = _ru(2 * roff, 1024)
    hn = jnp.zeros((tt, dh), jnp.float32)
    hn = hn.at[:n_i].set(inv_h).at[roff:roff + n_a].set(asset_h)

    pe = _ru(n_e, NS * EB)
    pad = pe - n_e
    tgt = edge_index[0].astype(jnp.int32)
    src = edge_index[1].astype(jnp.int32)
    tgt_p = jnp.concatenate([tgt, jnp.full((pad,), n_i, jnp.int32)])
    src_p = jnp.concatenate([src, jnp.full((pad,), n_a, jnp.int32)])
    srco_p = src_p + roff
    wi_p = jnp.concatenate([inv_norm_w, jnp.zeros((pad,), jnp.float32)])
    wa_p = jnp.concatenate([asset_norm_w, jnp.zeros((pad,), jnp.float32)])

    NP = 2
    hs = roff // NP
    ha = _ru(hs + 1, NS * 8)
    hd = _ru(hs // 8 + 1, NS * 8)
    out0 = jnp.stack([tgt_p, src_p])
    ohs, drows, dcbs = [], [], []
    for k in range(NP):
        inr = (out0 >= k * hs) & (out0 < (k + 1) * hs)
        tk = out0 - k * hs
        ohs.append(jnp.where(inr, tk, hs).reshape(NC, -1, 1, EB))
        drows.append(jnp.where(inr, tk // 8, hd - 1).reshape(NC, -1, 1, EB))
        dcbs.append(jnp.where(inr, (tk % 8) * 16, 0)
                    .astype(jnp.float32).reshape(NC, -1, 1, EB))

    nblk = pe // EB
    qidx = jnp.stack([tgt_p, srco_p]).reshape(NC, nblk, 1, EB)
    kvidx = jnp.stack([srco_p, tgt_p]).reshape(NC, nblk, 1, EB)
    meta = jnp.concatenate([qidx, kvidx] + ohs + drows, axis=2)
    wdat = jnp.concatenate(
        [jnp.stack([wi_p, wa_p]).reshape(NC, nblk, 1, EB)] + dcbs, axis=2)

    q, k, v = _qkv(hn, msg_W1, msg_b1, msg_W2, msg_b2, Wq, Wk, Wv, blk=1024)
    kv = jnp.concatenate([k, v], axis=1)

    num, den = _sc_edge(q, kv, meta, wdat, ha, hd, n_heads, d_k,
                        blocks_per_w=nblk // NS, n_passes=NP)

    numf = jnp.zeros((tt, dh), jnp.float32)
    denf = jnp.zeros((tt, 16), jnp.float32)
    for c in range(NC):
        base = c * roff
        for k in range(NP):
            den16 = den[c, k, :hs // 8].reshape(hs, 16)
            numf = numf.at[base + k * hs:base + (k + 1) * hs].set(
                num[c, k, :hs])
            denf = denf.at[base + k * hs:base + (k + 1) * hs].set(den16)

    out = _update(hn, numf, denf, upd_W1, upd_b1, upd_W2, upd_b2,
                  n_heads, d_k, blk=1024)
    return out[:n_i], out[roff:roff + n_a]


# speedup vs baseline: 1.9405x; 1.0089x over previous
import functools
import math

import jax
import jax.numpy as jnp
from jax import lax
from jax.experimental import pallas as pl
from jax.experimental.pallas import tpu as pltpu
from jax.experimental.pallas import tpu_sc as plsc

NC = 2
NS = 16
EB = 64


def _ru(x, m):
    return (x + m - 1) // m * m


def _gelu(x):
    return 0.5 * x * (1.0 + lax.erf(x * (1.0 / math.sqrt(2.0))))


def _qkv_body(h_ref, w1_ref, b1_ref, w2_ref, b2_ref, wq_ref, wk_ref, wv_ref,
              q_ref, k_ref, v_ref):
    h = h_ref[...]
    m = _gelu(jnp.dot(h, w1_ref[...], preferred_element_type=jnp.float32)
              + b1_ref[...])
    m = _gelu(jnp.dot(m, w2_ref[...], preferred_element_type=jnp.float32)
              + b2_ref[...])
    q_ref[...] = jnp.dot(h, wq_ref[...], preferred_element_type=jnp.float32)
    k_ref[...] = jnp.dot(h, wk_ref[...], preferred_element_type=jnp.float32)
    v_ref[...] = jnp.dot(m, wv_ref[...], preferred_element_type=jnp.float32)


def _qkv(hn, msg_W1, msg_b1, msg_W2, msg_b2, Wq, Wk, Wv, blk):
    tt, dh = hn.shape
    grid = (tt // blk,)
    full = lambda s: pl.BlockSpec(s, lambda i: (0, 0))
    rows = pl.BlockSpec((blk, dh), lambda i: (i, 0))
    return pl.pallas_call(
        _qkv_body,
        grid=grid,
        in_specs=[rows, full((dh, dh)), full((1, dh)), full((dh, dh)),
                  full((1, dh)), full((dh, dh)), full((dh, dh)),
                  full((dh, dh))],
        out_specs=[rows, rows, rows],
        out_shape=[jax.ShapeDtypeStruct((tt, dh), jnp.float32)] * 3,
    )(hn, msg_W1, msg_b1.reshape(1, dh), msg_W2, msg_b2.reshape(1, dh),
      Wq, Wk, Wv)


def _sc_edge(qtab, kvtab, meta, wdat, ha, hd, n_heads, d_k, blocks_per_w,
             n_passes):
    dh = qtab.shape[1]
    rp = ha // NS
    rpd = hd // NS
    inv_sqrt = 1.0 / math.sqrt(d_k)
    mesh = plsc.VectorSubcoreMesh(core_axis_name="c", subcore_axis_name="s",
                                  num_cores=NC, num_subcores=NS)

    @functools.partial(
        pl.kernel,
        out_type=[jax.ShapeDtypeStruct((NC, n_passes, ha, dh), jnp.float32),
                  jax.ShapeDtypeStruct((NC, n_passes, hd, dh), jnp.float32)],
        mesh=mesh,
        compiler_params=pltpu.CompilerParams(needs_layout_passes=False),
        scratch_types=[
            pltpu.VMEM((2 + 2 * n_passes, EB), jnp.int32),
            pltpu.VMEM((1 + n_passes, EB), jnp.float32),
            pltpu.VMEM((EB, dh), jnp.float32),
            pltpu.VMEM((EB, 2 * dh), jnp.float32),
            pltpu.VMEM((EB, dh), jnp.float32),
            pltpu.VMEM_SHARED((ha, dh), jnp.float32),
            pltpu.VMEM_SHARED((hd, dh), jnp.float32),
            pltpu.SemaphoreType.DMA,
            pltpu.SemaphoreType.DMA,
            pltpu.SemaphoreType.DMA,
            pltpu.SemaphoreType.DMA,
            pltpu.SemaphoreType.DMA,
            pltpu.SemaphoreType.DMA,
        ],
    )
    def body(qtab_h, kvtab_h, meta_h, wdat_h, znum_h, num_out, den_out,
             meta_v, wdat_v, qb, kvb, pb, num_acc, den_acc, sem0, sem1,
             semm0, semm1, sems0, sems1):
        c = lax.axis_index("c")
        s = lax.axis_index("s")
        pltpu.sync_copy(znum_h.at[pl.ds(0, EB)], pb)
        lane = lax.iota(jnp.int32, 16)

        for half in range(n_passes):
            pltpu.sync_copy(znum_h.at[pl.ds(s * rp, rp)],
                            num_acc.at[pl.ds(s * rp, rp)])
            pltpu.sync_copy(znum_h.at[pl.ds(s * rpd, rpd)],
                            den_acc.at[pl.ds(s * rpd, rpd)])
            plsc.subcore_barrier()

            def block_body(b, _):
                blk = s * blocks_per_w + b
                cpm = pltpu.async_copy(meta_h.at[c, blk], meta_v, semm0)
                cpw = pltpu.async_copy(wdat_h.at[c, blk], wdat_v, semm1)
                cpm.wait()
                cpw.wait()
                cp0 = pltpu.async_copy(qtab_h.at[meta_v.at[0]], qb, sem0)
                cp1 = pltpu.async_copy(kvtab_h.at[meta_v.at[1]], kvb, sem1)
                cp0.wait()
                cp1.wait()

                def group_body(g, _):
                    erow = lane + g * 16
                    wvec = plsc.load_gather(wdat_v,
                                            [jnp.zeros((16,), jnp.int32),
                                             erow])
                    cb = plsc.load_gather(wdat_v,
                                          [jnp.full((16,), 1 + half,
                                                    jnp.int32), erow]
                                          ).astype(jnp.int32)
                    wclip = jnp.maximum(wvec, 1e-10)
                    ps = []
                    for h in range(n_heads):
                        def dot_f(f4, acc):
                            f = f4 * 4
                            base = jnp.full((16,), h * d_k, jnp.int32) + f
                            for j in range(4):
                                qv = plsc.load_gather(qb, [erow, base + j])
                                kv = plsc.load_gather(kvb, [erow, base + j])
                                acc = acc + qv * kv
                            return acc
                        acc = lax.fori_loop(0, d_k // 4, dot_f,
                                            jnp.zeros((16,), jnp.float32))
                        p = wclip * jnp.exp(acc * inv_sqrt)
                        ps.append(p)
                        plsc.store_scatter(pb, [erow, cb + h], p)
                    for h in range(n_heads):
                        def wv_f(f4, _):
                            f = f4 * 4
                            base = jnp.full((16,), h * d_k, jnp.int32) + f
                            for j in range(4):
                                vv = plsc.load_gather(kvb,
                                                      [erow, base + j + dh])
                                plsc.store_scatter(qb, [erow, base + j],
                                                   vv * ps[h])
                            return 0
                        lax.fori_loop(0, d_k // 4, wv_f, 0)
                    return 0

                lax.fori_loop(0, EB // 16, group_body, 0)
                cpn = pltpu.async_copy(qb, num_acc.at[meta_v.at[2 + half]],
                                       sems0, add=True)
                cpd = pltpu.async_copy(
                    pb, den_acc.at[meta_v.at[2 + n_passes + half]], sems1,
                    add=True)
                cpn.wait()
                cpd.wait()

                def zero_body(g, _):
                    erow = lane + g * 16
                    cb = plsc.load_gather(wdat_v,
                                          [jnp.full((16,), 1 + half,
                                                    jnp.int32), erow]
                                          ).astype(jnp.int32)
                    zz = jnp.zeros((16,), jnp.float32)
                    for h in range(n_heads):
                        plsc.store_scatter(pb, [erow, cb + h], zz)
                    return 0

                lax.fori_loop(0, EB // 16, zero_body, 0)
                return 0

            lax.fori_loop(0, blocks_per_w, block_body, 0)
            plsc.subcore_barrier()
            pltpu.sync_copy(num_acc.at[pl.ds(s * rp, rp)],
                            num_out.at[c, half, pl.ds(s * rp, rp)])
            pltpu.sync_copy(den_acc.at[pl.ds(s * rpd, rpd)],
                            den_out.at[c, half, pl.ds(s * rpd, rpd)])

    znum = jnp.zeros((ha, dh), jnp.float32)
    return body(qtab, kvtab, meta, wdat, znum)


def _upd_body(h_ref, num_ref, den_ref, w1a_ref, w1b_ref, b1_ref, w2_ref,
              b2_ref, out_ref, *, n_heads, d_k):
    den16 = den_ref[...]
    heads = lax.broadcasted_iota(jnp.int32, (16, n_heads * d_k), 0)
    cols = lax.broadcasted_iota(jnp.int32, (16, n_heads * d_k), 1) // d_k
    expand = (heads == cols).astype(jnp.float32)
    den_exp = jnp.dot(den16, expand, preferred_element_type=jnp.float32)
    msg = num_ref[...] / (den_exp + 1e-10)
    h = h_ref[...]
    h1 = _gelu(jnp.dot(h, w1a_ref[...], preferred_element_type=jnp.float32)
               + jnp.dot(msg, w1b_ref[...], preferred_element_type=jnp.float32)
               + b1_ref[...])
    out_ref[...] = _gelu(jnp.dot(h1, w2_ref[...],
                                 preferred_element_type=jnp.float32)
                         + b2_ref[...])


def _update(hn, num, den, upd_W1, upd_b1, upd_W2, upd_b2, n_heads, d_k, blk):
    tt, dh = hn.shape
    grid = (tt // blk,)
    full = lambda s: pl.BlockSpec(s, lambda i: (0, 0))
    rows = pl.BlockSpec((blk, dh), lambda i: (i, 0))
    rows16 = pl.BlockSpec((blk, 16), lambda i: (i, 0))
    w1a = upd_W1[:dh]
    w1b = upd_W1[dh:]
    return pl.pallas_call(
        functools.partial(_upd_body, n_heads=n_heads, d_k=d_k),
        grid=grid,
        in_specs=[rows, rows, rows16, full((dh, dh)), full((dh, dh)),
                  full((1, dh)), full((dh, dh)), full((1, dh))],
        out_specs=rows,
        out_shape=jax.ShapeDtypeStruct((tt, dh), jnp.float32),
    )(hn, num, den, w1a, w1b, upd_b1.reshape(1, dh), upd_W2,
      upd_b2.reshape(1, dh))


def kernel(inv_h, asset_h, edge_index, inv_norm_w, asset_norm_w,
           num_investors, num_assets,
           msg_W1, msg_b1, msg_W2, msg_b2, Wq, Wk, Wv,
           upd_W1, upd_b1, upd_W2, upd_b2):
    n_i, dh = inv_h.shape
    n_a = asset_h.shape[0]
    n_e = edge_index.shape[1]
    hd = Wq.shape[1]
    d_k = 32
    n_heads = hd // d_k

    roff = _ru(max(n_i, n_a) + 1, NS * 8)
    tt = _ru(2 * roff, 1024)
    hn = jnp.zeros((tt, dh), jnp.float32)
    hn = hn.at[:n_i].set(inv_h).at[roff:roff + n_a].set(asset_h)

    pe = _ru(n_e, NS * EB)
    pad = pe - n_e
    tgt = edge_index[0].astype(jnp.int32)
    src = edge_index[1].astype(jnp.int32)
    tgt_p = jnp.concatenate([tgt, jnp.full((pad,), n_i, jnp.int32)])
    src_p = jnp.concatenate([src, jnp.full((pad,), n_a, jnp.int32)])
    srco_p = src_p + roff
    wi_p = jnp.concatenate([inv_norm_w, jnp.zeros((pad,), jnp.float32)])
    wa_p = jnp.concatenate([asset_norm_w, jnp.zeros((pad,), jnp.float32)])

    NP = 2
    hs = roff // NP
    ha = _ru(hs + 1, NS * 8)
    hd = _ru(hs // 8 + 1, NS * 8)
    out0 = jnp.stack([tgt_p, src_p])
    ohs, drows, dcbs = [], [], []
    for k in range(NP):
        inr = (out0 >= k * hs) & (out0 < (k + 1) * hs)
        tk = out0 - k * hs
        ohs.append(jnp.where(inr, tk, hs).reshape(NC, -1, 1, EB))
        drows.append(jnp.where(inr, tk // 8, hd - 1).reshape(NC, -1, 1, EB))
        dcbs.append(jnp.where(inr, (tk % 8) * 16, 0)
                    .astype(jnp.float32).reshape(NC, -1, 1, EB))

    nblk = pe // EB
    qidx = jnp.stack([tgt_p, srco_p]).reshape(NC, nblk, 1, EB)
    kvidx = jnp.stack([srco_p, tgt_p]).reshape(NC, nblk, 1, EB)
    meta = jnp.concatenate([qidx, kvidx] + ohs + drows, axis=2)
    wdat = jnp.concatenate(
        [jnp.stack([wi_p, wa_p]).reshape(NC, nblk, 1, EB)] + dcbs, axis=2)

    q, k, v = _qkv(hn, msg_W1, msg_b1, msg_W2, msg_b2, Wq, Wk, Wv, blk=1024)
    kv = jnp.concatenate([k, v], axis=1)

    num, den = _sc_edge(q, kv, meta, wdat, ha, hd, n_heads, d_k,
                        blocks_per_w=nblk // NS, n_passes=NP)

    numf = jnp.zeros((tt, dh), jnp.float32)
    denf = jnp.zeros((tt, 16), jnp.float32)
    for c in range(NC):
        base = c * roff
        for k in range(NP):
            den16 = den[c, k, :hs // 8].reshape(hs, 16)
            numf = numf.at[base + k * hs:base + (k + 1) * hs].set(
                num[c, k, :hs])
            denf = denf.at[base + k * hs:base + (k + 1) * hs].set(den16)

    out = _update(hn, numf, denf, upd_W1, upd_b1, upd_W2, upd_b2,
                  n_heads, d_k, blk=1024)
    return out[:n_i], out[roff:roff + n_a]
